# baseline (speedup 1.0000x reference)
import functools

import jax
import jax.numpy as jnp
from jax import lax
from jax.experimental import pallas as pl
from jax.experimental.pallas import tpu as pltpu
from jax.experimental.pallas import tpu_sc as plsc

BATCH = 16384
HIST = 50
D_MODEL = 64

_NC = 2
_NS = 16
_NW = _NC * _NS

_BB = 128
_NBLK = HIST * (BATCH // _BB)
_BLK_PER_W = _NBLK // _NW
_IDX_PER_W = _BLK_PER_W * _BB
_NBUF = 3
_K = 2


def _make_gather():
    mesh = plsc.VectorSubcoreMesh(core_axis_name="c", subcore_axis_name="s")

    @functools.partial(
        pl.kernel,
        mesh=mesh,
        out_type=jax.ShapeDtypeStruct((HIST, 8, BATCH // _BB, 8, 128), jnp.float32),
        scratch_types=[
            pltpu.VMEM((_IDX_PER_W,), jnp.int32),
            pltpu.VMEM((_NBUF, _BB, 128), jnp.float32),
            pltpu.VMEM((_NBUF, 8, 8, 128), jnp.float32),
            pltpu.SemaphoreType.DMA((_NBUF,)),
            pltpu.SemaphoreType.DMA((_NBUF,)),
        ],
        compiler_params=pltpu.CompilerParams(
            use_tc_tiling_on_sc=False, needs_layout_passes=False
        ),
    )
    def gather_kernel(idx_hbm, tbl_hbm, out_hbm, idx_v, rows_v, tbuf_v, gsem, osem):
        wid = lax.axis_index("s") * _NC + lax.axis_index("c")
        base_blk = wid * _BLK_PER_W
        pltpu.sync_copy(idx_hbm.at[pl.ds(base_blk * _BB, _IDX_PER_W)], idx_v)

        def start_gather(j, bj):
            pltpu.async_copy(
                tbl_hbm.at[idx_v.at[pl.ds(j * _BB, _BB)]],
                rows_v.at[bj],
                gsem.at[bj],
            )

        for j in range(_K):
            start_gather(j, j)

        row_idx = [lax.iota(jnp.int32, 16) + kg * 16 for kg in range(8)]

        def body(t, carry):
            b = lax.rem(t, _NBUF)
            n = base_blk + t
            h = n // (BATCH // _BB)
            b1 = lax.rem(n, BATCH // _BB)
            pltpu.make_async_copy(
                tbl_hbm.at[idx_v.at[pl.ds(0, _BB)]], rows_v.at[b], gsem.at[b]
            ).wait()

            rows = rows_v.at[b]
            tbuf = tbuf_v.at[b]

            def tr_body(d, carry2):
                col_idx = jnp.full((16,), d, jnp.int32)
                d1 = d // 8
                d0 = lax.rem(d, 8)
                for kg in range(8):
                    v = plsc.load_gather(rows, [row_idx[kg], col_idx])
                    tbuf[d1, d0, pl.ds(kg * 16, 16)] = v
                return carry2

            lax.fori_loop(0, D_MODEL, tr_body, 0)

            pltpu.async_copy(tbuf, out_hbm.at[h, :, b1, :, :], osem.at[b])

            j = t + _K

            @pl.when(j < _BLK_PER_W)
            def _():
                bj = lax.rem(j, _NBUF)

                @pl.when(j >= _NBUF)
                def _():
                    pltpu.make_async_copy(
                        tbuf_v.at[bj], out_hbm.at[0, :, 0, :, :], osem.at[bj]
                    ).wait()

                start_gather(j, bj)

            return carry

        lax.fori_loop(0, _BLK_PER_W, body, 0)

        for i in range(_NBUF):
            g = _BLK_PER_W - _NBUF + i
            b = g % _NBUF
            pltpu.make_async_copy(
                tbuf_v.at[b], out_hbm.at[0, :, 0, :, :], osem.at[b]
            ).wait()

    return gather_kernel


_gather = _make_gather()


@jax.jit
def kernel(player_id, table):
    idx_t = player_id.T.reshape(HIST * BATCH).astype(jnp.int32)
    tbl_pad = jnp.pad(table, ((0, 0), (0, 128 - D_MODEL)))
    o5 = _gather(idx_t, tbl_pad)
    out = o5.transpose(2, 4, 0, 1, 3).reshape(BATCH, HIST, D_MODEL)
    return out


# speedup vs baseline: 1.3995x; 1.3995x over previous
import functools

import jax
import jax.numpy as jnp
from jax import lax
from jax.experimental import pallas as pl
from jax.experimental.pallas import tpu as pltpu
from jax.experimental.pallas import tpu_sc as plsc

BATCH = 16384
HIST = 50
D_MODEL = 64
B_FLAT = BATCH * HIST

_NC = 2
_NS = 16
_NW = _NC * _NS

_B_PER_W = B_FLAT // _NW
_CHUNK = 256
_N_CHUNKS = _B_PER_W // _CHUNK
_NBUF = 5
_K = 3


def _make_gather():
    mesh = plsc.VectorSubcoreMesh(core_axis_name="c", subcore_axis_name="s")

    @functools.partial(
        pl.kernel,
        mesh=mesh,
        out_type=jax.ShapeDtypeStruct((B_FLAT, D_MODEL), jnp.float32),
        scratch_types=[
            pltpu.VMEM((_B_PER_W,), jnp.int32),
            pltpu.VMEM((_NBUF, _CHUNK, D_MODEL), jnp.float32),
            pltpu.SemaphoreType.DMA((_NBUF,)),
            pltpu.SemaphoreType.DMA((_NBUF,)),
        ],
        compiler_params=pltpu.CompilerParams(use_tc_tiling_on_sc=False),
    )
    def gather_kernel(idx_hbm, table_hbm, out_hbm, idx_v, rows_v, gsem, osem):
        wid = lax.axis_index("s") * _NC + lax.axis_index("c")
        base = wid * _B_PER_W
        pltpu.sync_copy(idx_hbm.at[pl.ds(base, _B_PER_W)], idx_v)

        def start_gather(j, bj):
            pltpu.async_copy(
                table_hbm.at[idx_v.at[pl.ds(j * _CHUNK, _CHUNK)]],
                rows_v.at[bj],
                gsem.at[bj],
            )

        for j in range(_K):
            start_gather(j, j)

        def body(g, carry):
            b = lax.rem(g, _NBUF)
            pltpu.make_async_copy(
                table_hbm.at[idx_v.at[pl.ds(0, _CHUNK)]], rows_v.at[b], gsem.at[b]
            ).wait()
            pltpu.async_copy(
                rows_v.at[b],
                out_hbm.at[pl.ds(base + g * _CHUNK, _CHUNK)],
                osem.at[b],
            )
            j = g + _K

            @pl.when(j < _N_CHUNKS)
            def _():
                bj = lax.rem(j, _NBUF)

                @pl.when(j >= _NBUF)
                def _():
                    pltpu.make_async_copy(
                        rows_v.at[bj], out_hbm.at[pl.ds(base, _CHUNK)], osem.at[bj]
                    ).wait()

                start_gather(j, bj)

            return carry

        lax.fori_loop(0, _N_CHUNKS, body, 0)

        for i in range(_NBUF):
            g = _N_CHUNKS - _NBUF + i
            b = g % _NBUF
            pltpu.make_async_copy(
                rows_v.at[b], out_hbm.at[pl.ds(base, _CHUNK)], osem.at[b]
            ).wait()

    return gather_kernel


_gather = _make_gather()


@jax.jit
def kernel(player_id, table):
    idx_flat = player_id.reshape(B_FLAT).astype(jnp.int32)
    out = _gather(idx_flat, table)
    return out.reshape(BATCH, HIST, D_MODEL)
